# baseline (speedup 1.0000x reference)
import functools

import jax
import jax.numpy as jnp
from jax import lax
from jax.experimental import pallas as pl
from jax.experimental.pallas import tpu as pltpu
from jax.experimental.pallas import tpu_sc as plsc

VOCAB = 1000000
EMB = 64
BATCH = 4096
SEQ = 50

NUM_PAIRS = BATCH // 2
TOK_PER_PAIR = 2 * SEQ
NC, NS, L = 2, 16, 16
NW = NC * NS
PAIRS_PER_W = NUM_PAIRS // NW
NVR = EMB // L


def _sc_body(idx_hbm, w_hbm, b_hbm, table_hbm, out_hbm,
             idx_v, w_v, b_v, rows0_v, rows1_v, out_v, sem0, sem1):
    cid = lax.axis_index("c")
    sid = lax.axis_index("s")
    wid = sid * NC + cid
    base = wid * PAIRS_PER_W

    pltpu.sync_copy(idx_hbm.at[pl.ds(base, PAIRS_PER_W)], idx_v)
    pltpu.sync_copy(w_hbm, w_v)
    pltpu.sync_copy(b_hbm, b_v)

    wv = [w_v[pl.ds(v * L, L)] for v in range(2 * NVR)]
    sems = (sem0, sem1)
    rows = (rows0_v, rows1_v)

    def _gather(pair_i, buf):
        return pltpu.async_copy(
            table_hbm.at[idx_v.at[pair_i]], rows[buf], sems[buf])

    _gather(0, 0)
    _gather(1, 1)

    @pl.loop(0, PAIRS_PER_W, step=2)
    def _pair_loop(i):
        for buf in range(2):
            it = i + buf
            pltpu.make_async_copy(
                table_hbm.at[idx_v.at[it]], rows[buf], sems[buf]).wait()
            acc = [jnp.zeros((L,), jnp.float32) for _ in range(2 * NVR)]
            for j in range(SEQ):
                for v in range(NVR):
                    acc[v] = acc[v] + rows[buf][j, pl.ds(v * L, L)]
            for j in range(SEQ, TOK_PER_PAIR):
                for v in range(NVR):
                    acc[NVR + v] = acc[NVR + v] + rows[buf][j, pl.ds(v * L, L)]
            pair = acc[0] * wv[0]
            for v in range(1, 2 * NVR):
                pair = pair + acc[v] * wv[v]
            csum = plsc.cumsum(pair)
            lane15 = lax.iota(jnp.int32, L) == (L - 1)
            plsc.store_scatter(out_v, [jnp.full((L,), it, jnp.int32)],
                               csum, mask=lane15)

            @pl.when(it + 2 < PAIRS_PER_W)
            def _():
                _gather(it + 2, buf)

    inv = jnp.float32(1.0 / SEQ)
    bvec = b_v[...]
    for t in range(PAIRS_PER_W // L):
        x = out_v[pl.ds(t * L, L)] * inv + bvec
        out_v[pl.ds(t * L, L)] = 1.0 / (1.0 + jnp.exp(-x))

    pltpu.sync_copy(out_v, out_hbm.at[pl.ds(base, PAIRS_PER_W)])


@jax.jit
def _run(idx_pairs, table, w_flat, b_vec):
    mesh = plsc.VectorSubcoreMesh(core_axis_name="c", subcore_axis_name="s")
    k = functools.partial(
        pl.kernel,
        out_type=jax.ShapeDtypeStruct((NUM_PAIRS,), jnp.float32),
        mesh=mesh,
        compiler_params=pltpu.CompilerParams(
            needs_layout_passes=False, use_tc_tiling_on_sc=False),
        scratch_types=[
            pltpu.VMEM((PAIRS_PER_W, TOK_PER_PAIR), jnp.int32),
            pltpu.VMEM((2 * EMB,), jnp.float32),
            pltpu.VMEM((L,), jnp.float32),
            pltpu.VMEM((TOK_PER_PAIR, EMB), jnp.float32),
            pltpu.VMEM((TOK_PER_PAIR, EMB), jnp.float32),
            pltpu.VMEM((PAIRS_PER_W,), jnp.float32),
            pltpu.SemaphoreType.DMA,
            pltpu.SemaphoreType.DMA,
        ],
    )(_sc_body)
    return k(idx_pairs, w_flat, b_vec, table)


def kernel(indices, table, W, b):
    idx_pairs = indices.reshape(NUM_PAIRS, TOK_PER_PAIR).astype(jnp.int32)
    w_flat = W.reshape(2 * EMB).astype(jnp.float32)
    b_vec = jnp.broadcast_to(b.astype(jnp.float32), (L,))
    out = _run(idx_pairs, table, w_flat, b_vec)
    return out.reshape(NUM_PAIRS, 1)


# speedup vs baseline: 2.8844x; 2.8844x over previous
import functools

import jax
import jax.numpy as jnp
from jax import lax
from jax.experimental import pallas as pl
from jax.experimental.pallas import tpu as pltpu
from jax.experimental.pallas import tpu_sc as plsc

VOCAB = 1000000
EMB = 64
BATCH = 4096
SEQ = 50

NUM_PAIRS = BATCH // 2
NC, NS, L = 2, 16, 16
NW = NC * NS
COLS_PER_W = BATCH // NW
PAIRS_PER_W = COLS_PER_W // 2
CB = 4096


def _matmul_body(w2_ref, t_ref, out_ref):
    out_ref[...] = jnp.dot(w2_ref[...], t_ref[...],
                           preferred_element_type=jnp.float32)


def _token_weights(w2, table_t):
    grid = (VOCAB + CB - 1) // CB
    return pl.pallas_call(
        _matmul_body,
        grid=(grid,),
        in_specs=[
            pl.BlockSpec((2, EMB), lambda i: (0, 0)),
            pl.BlockSpec((EMB, CB), lambda i: (0, i)),
        ],
        out_specs=pl.BlockSpec((2, CB), lambda i: (0, i)),
        out_shape=jax.ShapeDtypeStruct((2, VOCAB), jnp.float32),
    )(w2, table_t)


def _sc_body(idxt_hbm, tw_hbm, b_hbm, out_hbm,
             idx_v, vals_v, colsum_v, out_v, b_v, sem):
    cid = lax.axis_index("c")
    sid = lax.axis_index("s")
    wid = sid * NC + cid
    col0 = wid * COLS_PER_W

    pltpu.sync_copy(idxt_hbm.at[:, pl.ds(col0, COLS_PER_W)], idx_v)
    pltpu.sync_copy(b_hbm, b_v)

    off = (lax.iota(jnp.int32, L) % 2) * VOCAB
    for j in range(SEQ):
        for g in range(COLS_PER_W // L):
            sl = pl.ds(g * L, L)
            idx_v[j, sl] = idx_v[j, sl] + off

    @pl.loop(0, SEQ)
    def _fire(j):
        pltpu.async_copy(tw_hbm.at[idx_v.at[j]], vals_v.at[j], sem)

    @pl.loop(0, SEQ)
    def _drain(j):
        pltpu.make_async_copy(tw_hbm.at[idx_v.at[j]], vals_v.at[j], sem).wait()

    for g in range(COLS_PER_W // L):
        sl = pl.ds(g * L, L)
        acc = vals_v[0, sl]
        for j in range(1, SEQ):
            acc = acc + vals_v[j, sl]
        colsum_v[sl] = acc

    ev = lax.iota(jnp.int32, L) * 2
    od = ev + 1
    bvec = b_v[...]
    inv = jnp.float32(1.0 / SEQ)
    for m in range(PAIRS_PER_W // L):
        base = jnp.full((L,), 2 * L * m, jnp.int32)
        evens = plsc.load_gather(colsum_v, [base + ev])
        odds = plsc.load_gather(colsum_v, [base + od])
        x = (evens + odds) * inv + bvec
        out_v[pl.ds(m * L, L)] = 1.0 / (1.0 + jnp.exp(-x))

    pltpu.sync_copy(out_v, out_hbm.at[pl.ds(wid * PAIRS_PER_W, PAIRS_PER_W)])


def _gather_pool(idx_t, tw_flat, b_vec):
    mesh = plsc.VectorSubcoreMesh(core_axis_name="c", subcore_axis_name="s")
    return functools.partial(
        pl.kernel,
        out_type=jax.ShapeDtypeStruct((NUM_PAIRS,), jnp.float32),
        mesh=mesh,
        compiler_params=pltpu.CompilerParams(
            needs_layout_passes=False, use_tc_tiling_on_sc=False),
        scratch_types=[
            pltpu.VMEM((SEQ, COLS_PER_W), jnp.int32),
            pltpu.VMEM((SEQ, COLS_PER_W), jnp.float32),
            pltpu.VMEM((COLS_PER_W,), jnp.float32),
            pltpu.VMEM((PAIRS_PER_W,), jnp.float32),
            pltpu.VMEM((L,), jnp.float32),
            pltpu.SemaphoreType.DMA,
        ],
    )(_sc_body)(idx_t, tw_flat, b_vec)


@jax.jit
def _run(indices, table, W, b):
    idx_t = indices.T.astype(jnp.int32)
    table_t = table.T
    w2 = W.reshape(2, EMB)
    tw = _token_weights(w2, table_t)
    b_vec = jnp.broadcast_to(b.astype(jnp.float32), (L,))
    out = _gather_pool(idx_t, tw.reshape(2 * VOCAB), b_vec)
    return out.reshape(NUM_PAIRS, 1)


def kernel(indices, table, W, b):
    return _run(indices, table, W, b)


# speedup vs baseline: 5.0042x; 1.7349x over previous
import functools

import jax
import jax.numpy as jnp
from jax import lax
from jax.experimental import pallas as pl
from jax.experimental.pallas import tpu as pltpu
from jax.experimental.pallas import tpu_sc as plsc

VOCAB = 1000000
EMB = 64
BATCH = 4096
SEQ = 50

NUM_PAIRS = BATCH // 2
NC, NS, L = 2, 16, 16
NW = NC * NS
COLS_PER_W = BATCH // NW
PAIRS_PER_W = COLS_PER_W // 2
CB = 16384


def _matmul_body(w2_ref, t_ref, out_ref):
    out_ref[...] = jnp.dot(w2_ref[...], t_ref[...],
                           preferred_element_type=jnp.float32)


def _token_weights(w2, table_t):
    grid = (VOCAB + CB - 1) // CB
    return pl.pallas_call(
        _matmul_body,
        grid=(grid,),
        in_specs=[
            pl.BlockSpec((2, EMB), lambda i: (0, 0)),
            pl.BlockSpec((EMB, CB), lambda i: (0, i)),
        ],
        out_specs=pl.BlockSpec((2, CB), lambda i: (0, i)),
        out_shape=jax.ShapeDtypeStruct((2, VOCAB), jnp.float32),
    )(w2, table_t)


def _sc_body(idxt_hbm, tw_hbm, b_hbm, out_hbm,
             idx_v, vals_v, colsum_v, out_v, b_v, sem):
    cid = lax.axis_index("c")
    sid = lax.axis_index("s")
    wid = sid * NC + cid
    col0 = wid * COLS_PER_W

    pltpu.sync_copy(idxt_hbm.at[:, pl.ds(col0, COLS_PER_W)], idx_v)
    pltpu.sync_copy(b_hbm, b_v)

    off = (lax.iota(jnp.int32, L) % 2) * VOCAB
    for j in range(SEQ):
        for g in range(COLS_PER_W // L):
            sl = pl.ds(g * L, L)
            idx_v[j, sl] = idx_v[j, sl] + off

    @pl.loop(0, SEQ)
    def _fire(j):
        pltpu.async_copy(tw_hbm.at[idx_v.at[j]], vals_v.at[j], sem)

    @pl.loop(0, SEQ)
    def _drain(j):
        pltpu.make_async_copy(tw_hbm.at[idx_v.at[j]], vals_v.at[j], sem).wait()

    for g in range(COLS_PER_W // L):
        sl = pl.ds(g * L, L)
        acc = vals_v[0, sl]
        for j in range(1, SEQ):
            acc = acc + vals_v[j, sl]
        colsum_v[sl] = acc

    ev = lax.iota(jnp.int32, L) * 2
    od = ev + 1
    bvec = b_v[...]
    inv = jnp.float32(1.0 / SEQ)
    for m in range(PAIRS_PER_W // L):
        base = jnp.full((L,), 2 * L * m, jnp.int32)
        evens = plsc.load_gather(colsum_v, [base + ev])
        odds = plsc.load_gather(colsum_v, [base + od])
        x = (evens + odds) * inv + bvec
        out_v[pl.ds(m * L, L)] = 1.0 / (1.0 + jnp.exp(-x))

    pltpu.sync_copy(out_v, out_hbm.at[pl.ds(wid * PAIRS_PER_W, PAIRS_PER_W)])


def _gather_pool(idx_t, tw_flat, b_vec):
    mesh = plsc.VectorSubcoreMesh(core_axis_name="c", subcore_axis_name="s")
    return functools.partial(
        pl.kernel,
        out_type=jax.ShapeDtypeStruct((NUM_PAIRS,), jnp.float32),
        mesh=mesh,
        compiler_params=pltpu.CompilerParams(
            needs_layout_passes=False, use_tc_tiling_on_sc=False),
        scratch_types=[
            pltpu.VMEM((SEQ, COLS_PER_W), jnp.int32),
            pltpu.VMEM((SEQ, COLS_PER_W), jnp.float32),
            pltpu.VMEM((COLS_PER_W,), jnp.float32),
            pltpu.VMEM((PAIRS_PER_W,), jnp.float32),
            pltpu.VMEM((L,), jnp.float32),
            pltpu.SemaphoreType.DMA,
        ],
    )(_sc_body)(idx_t, tw_flat, b_vec)


@jax.jit
def _run(indices, table, W, b):
    idx_t = indices.T.astype(jnp.int32)
    table_t = table.T
    w2 = W.reshape(2, EMB)
    tw = _token_weights(w2, table_t)
    b_vec = jnp.broadcast_to(b.astype(jnp.float32), (L,))
    out = _gather_pool(idx_t, tw.reshape(2 * VOCAB), b_vec)
    return out.reshape(NUM_PAIRS, 1)


def kernel(indices, table, W, b):
    return _run(indices, table, W, b)


# speedup vs baseline: 5.3697x; 1.0730x over previous
import functools

import jax
import jax.numpy as jnp
from jax import lax
from jax.experimental import pallas as pl
from jax.experimental.pallas import tpu as pltpu
from jax.experimental.pallas import tpu_sc as plsc

VOCAB = 1000000
EMB = 64
BATCH = 4096
SEQ = 50

NUM_PAIRS = BATCH // 2
NC, NS, L = 2, 16, 16
NW = NC * NS
COLS_PER_W = BATCH // NW
PAIRS_PER_W = COLS_PER_W // 2
CB = 32768


def _matmul_body(w2_ref, t_ref, out_ref):
    out_ref[...] = jnp.dot(w2_ref[...], t_ref[...],
                           preferred_element_type=jnp.float32)


def _token_weights(w2, table_t):
    grid = (VOCAB + CB - 1) // CB
    return pl.pallas_call(
        _matmul_body,
        grid=(grid,),
        in_specs=[
            pl.BlockSpec((2, EMB), lambda i: (0, 0)),
            pl.BlockSpec((EMB, CB), lambda i: (0, i)),
        ],
        out_specs=pl.BlockSpec((2, CB), lambda i: (0, i)),
        out_shape=jax.ShapeDtypeStruct((2, VOCAB), jnp.float32),
    )(w2, table_t)


def _sc_body(idxt_hbm, tw_hbm, b_hbm, out_hbm,
             idx_v, vals_v, colsum_v, out_v, b_v, sem):
    cid = lax.axis_index("c")
    sid = lax.axis_index("s")
    wid = sid * NC + cid
    col0 = wid * COLS_PER_W

    pltpu.sync_copy(idxt_hbm.at[:, pl.ds(col0, COLS_PER_W)], idx_v)
    pltpu.sync_copy(b_hbm, b_v)

    off = (lax.iota(jnp.int32, L) % 2) * VOCAB
    for j in range(SEQ):
        for g in range(COLS_PER_W // L):
            sl = pl.ds(g * L, L)
            idx_v[j, sl] = idx_v[j, sl] + off

    @pl.loop(0, SEQ)
    def _fire(j):
        pltpu.async_copy(tw_hbm.at[idx_v.at[j]], vals_v.at[j], sem)

    @pl.loop(0, SEQ)
    def _drain(j):
        pltpu.make_async_copy(tw_hbm.at[idx_v.at[j]], vals_v.at[j], sem).wait()

    for g in range(COLS_PER_W // L):
        sl = pl.ds(g * L, L)
        acc = vals_v[0, sl]
        for j in range(1, SEQ):
            acc = acc + vals_v[j, sl]
        colsum_v[sl] = acc

    ev = lax.iota(jnp.int32, L) * 2
    od = ev + 1
    bvec = b_v[...]
    inv = jnp.float32(1.0 / SEQ)
    for m in range(PAIRS_PER_W // L):
        base = jnp.full((L,), 2 * L * m, jnp.int32)
        evens = plsc.load_gather(colsum_v, [base + ev])
        odds = plsc.load_gather(colsum_v, [base + od])
        x = (evens + odds) * inv + bvec
        out_v[pl.ds(m * L, L)] = 1.0 / (1.0 + jnp.exp(-x))

    pltpu.sync_copy(out_v, out_hbm.at[pl.ds(wid * PAIRS_PER_W, PAIRS_PER_W)])


def _gather_pool(idx_t, tw_flat, b_vec):
    mesh = plsc.VectorSubcoreMesh(core_axis_name="c", subcore_axis_name="s")
    return functools.partial(
        pl.kernel,
        out_type=jax.ShapeDtypeStruct((NUM_PAIRS,), jnp.float32),
        mesh=mesh,
        compiler_params=pltpu.CompilerParams(
            needs_layout_passes=False, use_tc_tiling_on_sc=False),
        scratch_types=[
            pltpu.VMEM((SEQ, COLS_PER_W), jnp.int32),
            pltpu.VMEM((SEQ, COLS_PER_W), jnp.float32),
            pltpu.VMEM((COLS_PER_W,), jnp.float32),
            pltpu.VMEM((PAIRS_PER_W,), jnp.float32),
            pltpu.VMEM((L,), jnp.float32),
            pltpu.SemaphoreType.DMA,
        ],
    )(_sc_body)(idx_t, tw_flat, b_vec)


@jax.jit
def _run(indices, table, W, b):
    idx_t = indices.T.astype(jnp.int32)
    table_t = table.T
    w2 = W.reshape(2, EMB)
    tw = _token_weights(w2, table_t)
    b_vec = jnp.broadcast_to(b.astype(jnp.float32), (L,))
    out = _gather_pool(idx_t, tw.reshape(2 * VOCAB), b_vec)
    return out.reshape(NUM_PAIRS, 1)


def kernel(indices, table, W, b):
    return _run(indices, table, W, b)
